# Initial kernel scaffold; baseline (speedup 1.0000x reference)
import jax
import jax.numpy as jnp
from jax.experimental import pallas as pl


def kernel(h_V, h_E, E_idx, X, Wp, bp, W1, b1, W2, b2, W3, b3, Wd1, bd1, Wd2, bd2, g0, be0, g1, be1):
    raise NotImplementedError("write your pallas kernel here")


# speedup vs baseline: 12.0237x; 12.0237x over previous
import functools

import jax
import jax.numpy as jnp
import numpy as np
from jax import lax
from jax.experimental import pallas as pl
from jax.experimental.pallas import tpu as pltpu
from jax.experimental.pallas import tpu_sc as plsc

B, L, K = 4, 1024, 32
ND, ED, HD, NP = 128, 128, 128, 8
POS_SCALE = 10.0
NIDX = B * L * K
BL = 64
E = BL * K
NBLK = (B * L) // BL

NC, NS = 2, 16
NW = NC * NS
NPW = NIDX // NW
CHUNK = 128
NCHUNK = NPW // CHUNK


def _sc_gather_body(hv_hbm, x_hbm, idx_hbm, g1_hbm, g2_hbm,
                    idx_v, rows1_v, rows2_v, sem1, sem2):
    cid = lax.axis_index("c")
    sid = lax.axis_index("s")
    wid = sid * NC + cid
    base = wid * NPW
    batch_off = (base // (L * K)) * L

    def chunk(i, carry):
        off = base + i * CHUNK
        pltpu.sync_copy(idx_hbm.at[pl.ds(off, CHUNK)], idx_v)
        for j in range(CHUNK // 16):
            sl = pl.ds(j * 16, 16)
            idx_v[sl] = idx_v[sl] + batch_off
        cp1 = pltpu.async_copy(hv_hbm.at[idx_v], rows1_v, sem1)
        cp2 = pltpu.async_copy(x_hbm.at[idx_v], rows2_v, sem2)
        cp1.wait()
        cp2.wait()
        pltpu.sync_copy(rows1_v, g1_hbm.at[pl.ds(off, CHUNK)])
        pltpu.sync_copy(rows2_v, g2_hbm.at[pl.ds(off, CHUNK)])
        return carry

    lax.fori_loop(0, NCHUNK, chunk, 0)


def _sc_gather(hv2, x16, idx_flat):
    mesh = plsc.VectorSubcoreMesh(core_axis_name="c", subcore_axis_name="s")
    fn = pl.kernel(
        _sc_gather_body,
        out_type=[
            jax.ShapeDtypeStruct((NIDX, ND), jnp.float32),
            jax.ShapeDtypeStruct((NIDX, 16), jnp.float32),
        ],
        mesh=mesh,
        compiler_params=pltpu.CompilerParams(use_tc_tiling_on_sc=False),
        scratch_types=[
            pltpu.VMEM((CHUNK,), jnp.int32),
            pltpu.VMEM((CHUNK, ND), jnp.float32),
            pltpu.VMEM((CHUNK, 16), jnp.float32),
            pltpu.SemaphoreType.DMA,
            pltpu.SemaphoreType.DMA,
        ],
    )
    return fn(hv2, x16, idx_flat)


def _nt(a, b):
    return lax.dot_general(a, b, (((1,), (1,)), ((), ())),
                           preferred_element_type=jnp.float32)


def _nn(a, b):
    return lax.dot_general(a, b, (((1,), (0,)), ((), ())),
                           preferred_element_type=jnp.float32)


def _tn(a, b):
    return lax.dot_general(a, b, (((0,), (0,)), ((), ())),
                           preferred_element_type=jnp.float32)


def _frames_t(xT):
    n_ = [xT[i:i + 1] for i in range(3)]
    ca = [xT[i:i + 1] for i in range(3, 6)]
    c_ = [xT[i:i + 1] for i in range(6, 9)]
    v1 = [c_[i] - ca[i] for i in range(3)]
    v2 = [n_[i] - ca[i] for i in range(3)]
    n1 = jnp.sqrt(v1[0] * v1[0] + v1[1] * v1[1] + v1[2] * v1[2])
    e1 = [v1[i] / (n1 + 1e-8) for i in range(3)]
    d = e1[0] * v2[0] + e1[1] * v2[1] + e1[2] * v2[2]
    u2 = [v2[i] - e1[i] * d for i in range(3)]
    n2 = jnp.sqrt(u2[0] * u2[0] + u2[1] * u2[1] + u2[2] * u2[2])
    e2 = [u2[i] / (n2 + 1e-8) for i in range(3)]
    e3 = [e1[1] * e2[2] - e1[2] * e2[1],
          e1[2] * e2[0] - e1[0] * e2[2],
          e1[0] * e2[1] - e1[1] * e2[0]]
    t = [ca[i] * (1.0 / POS_SCALE) for i in range(3)]
    return e1, e2, e3, t


def _layer_norm_t(x, g, b):
    mu = jnp.mean(x, axis=0, keepdims=True)
    var = jnp.mean((x - mu) ** 2, axis=0, keepdims=True)
    return (x - mu) / jnp.sqrt(var + 1e-5) * g + b


def _tc_body(hv_ref, x_ref, he_ref, g1_ref, g2_ref,
             wp_ref, bp_ref, w1n_ref, w1e_ref, w1nb_ref, w1gn_ref,
             w1ge_ref, b1_ref, w2_ref, b2_ref, w3_ref, b3_ref,
             wd1_ref, bd1_ref, wd2_ref, bd2_ref,
             g0_ref, be0_ref, g1w_ref, be1_ref,
             i16_ref, i128_ref, out_ref):
    hv = hv_ref[...]
    i16 = i16_ref[...]
    i128 = i128_ref[...]

    xT = _nt(i16, x_ref[...])
    e1, e2, e3, t = _frames_t(xT)
    plT = _nt(wp_ref[...], hv) + bp_ref[...]
    plx, ply, plz = plT[0:8], plT[8:16], plT[16:24]
    pln = jnp.sqrt(plx * plx + ply * ply + plz * plz + 1e-8)
    pg = [e1[i] * plx + e2[i] * ply + e3[i] * plz + t[i] for i in range(3)]
    nodegeo = jnp.concatenate([plx, ply, plz, pln], axis=0)
    aT = _nt(w1n_ref[...], hv) + _nn(w1gn_ref[...], nodegeo) + b1_ref[...]

    dmat = jnp.concatenate(pg + t + e1 + e2 + e3, axis=0)

    il = lax.broadcasted_iota(jnp.int32, (BL, E), 0)
    ie = lax.broadcasted_iota(jnp.int32, (BL, E), 1)
    sel = jnp.where(lax.shift_right_logical(ie, 5) == il, 1.0, 0.0)
    sel = sel.astype(jnp.float32)

    de = _nn(dmat, sel)
    pgxe, pgye, pgze = de[0:8], de[8:16], de[16:24]
    txe, tye, tze = de[24:25], de[25:26], de[26:27]
    e1xe, e1ye, e1ze = de[27:28], de[28:29], de[29:30]
    e2xe, e2ye, e2ze = de[30:31], de[31:32], de[32:33]
    e3xe, e3ye, e3ze = de[33:34], de[34:35], de[35:36]

    g1 = g1_ref[...]
    g2T = _nt(i16, g2_ref[...])
    ne1, ne2, ne3, nt_ = _frames_t(g2T)
    nplT = _nt(wp_ref[...], g1) + bp_ref[...]
    npx, npy, npz = nplT[0:8], nplT[8:16], nplT[16:24]
    pgn = [ne1[i] * npx + ne2[i] * npy + ne3[i] * npz + nt_[i]
           for i in range(3)]

    dx = pgn[0] - txe
    dy = pgn[1] - tye
    dz = pgn[2] - tze
    nblx = e1xe * dx + e1ye * dy + e1ze * dz
    nbly = e2xe * dx + e2ye * dy + e2ze * dz
    nblz = e3xe * dx + e3ye * dy + e3ze * dz
    nbln = jnp.sqrt(nblx * nblx + nbly * nbly + nblz * nblz + 1e-8)
    gx = pgxe - pgn[0]
    gy = pgye - pgn[1]
    gz = pgze - pgn[2]
    nbgn = jnp.sqrt(gx * gx + gy * gy + gz * gz + 1e-8)
    geomT = jnp.concatenate([nblx, nbly, nblz, nbln, nbgn], axis=0)

    h1 = _nt(w1e_ref[...], he_ref[...])
    h1 = h1 + _nt(w1nb_ref[...], g1)
    h1 = h1 + _nn(w1ge_ref[...], geomT)
    h1 = h1 + _nn(aT, sel)
    h1 = jnp.maximum(h1, 0.0)
    h2 = jnp.maximum(_nn(w2_ref[...], h1) + b2_ref[...], 0.0)
    mT = _nn(w3_ref[...], h2) + b3_ref[...]

    meanT = _nt(mT, sel) * (1.0 / K)

    hvT = _nt(i128, hv)
    r0 = _layer_norm_t(hvT + meanT, g0_ref[...], be0_ref[...])
    d1 = jnp.maximum(_nn(wd1_ref[...], r0) + bd1_ref[...], 0.0)
    dm = _nn(wd2_ref[...], d1) + bd2_ref[...]
    r1 = _layer_norm_t(r0 + dm, g1w_ref[...], be1_ref[...])

    out_ref[...] = _tn(r1, i128)


def _tc_forward(hv2, x16, he2, g1, g2, wp_p, bp_c, w1n, w1e, w1nb, w1gn,
                w1ge, b1c, w2, b2c, w3, b3c, wd1, bd1c, wd2, bd2c,
                g0c, be0c, g1c_, be1c, interpret=False):
    i16 = jnp.eye(16, dtype=jnp.float32)
    i128 = jnp.eye(128, dtype=jnp.float32)

    def row_spec(r, c):
        return pl.BlockSpec((r, c), lambda i: (i, 0))

    def full_spec(shape):
        return pl.BlockSpec(shape, lambda i: tuple(0 for _ in shape))

    in_specs = [
        row_spec(BL, ND),
        row_spec(BL, 16),
        row_spec(E, ED),
        row_spec(E, ND),
        row_spec(E, 16),
        full_spec((24, ND)),
        full_spec((24, 1)),
        full_spec((HD, ND)),
        full_spec((HD, ED)),
        full_spec((HD, ND)),
        full_spec((HD, 32)),
        full_spec((HD, 40)),
        full_spec((HD, 1)),
        full_spec((HD, HD)),
        full_spec((HD, 1)),
        full_spec((HD, HD)),
        full_spec((HD, 1)),
        full_spec((4 * HD, HD)),
        full_spec((4 * HD, 1)),
        full_spec((HD, 4 * HD)),
        full_spec((HD, 1)),
        full_spec((HD, 1)),
        full_spec((HD, 1)),
        full_spec((HD, 1)),
        full_spec((HD, 1)),
        full_spec((16, 16)),
        full_spec((ND, ND)),
    ]
    out = pl.pallas_call(
        _tc_body,
        grid=(NBLK,),
        in_specs=in_specs,
        out_specs=pl.BlockSpec((BL, ND), lambda i: (i, 0)),
        out_shape=jax.ShapeDtypeStruct((B * L, ND), jnp.float32),
        interpret=interpret,
    )(hv2, x16, he2, g1, g2, wp_p, bp_c, w1n, w1e, w1nb, w1gn, w1ge,
      b1c, w2, b2c, w3, b3c, wd1, bd1c, wd2, bd2c, g0c, be0c, g1c_,
      be1c, i16, i128)
    return out


_PERM24 = np.array([3 * p + c for c in range(3) for p in range(NP)])


def _prep(h_V, h_E, E_idx, X, Wp, bp, W1, b1, W2, b2, W3, b3,
          Wd1, bd1, Wd2, bd2, g0, be0, g1, be1):
    hv2 = h_V.reshape(B * L, ND)
    x9 = X[:, :, :3, :].reshape(B * L, 9)
    x16 = jnp.concatenate(
        [x9, jnp.zeros((B * L, 7), jnp.float32)], axis=1)
    he2 = h_E.reshape(NIDX, ED)
    idx_flat = E_idx.reshape(NIDX).astype(jnp.int32)

    wp_p = Wp[_PERM24, :]
    bp_c = bp[_PERM24].reshape(24, 1)
    w1n = W1[:, 0:ND]
    w1e = W1[:, ND:ND + ED]
    w1nb = W1[:, ND + ED:2 * ND + ED]
    base = 2 * ND + ED
    w1pl = W1[:, base:base + 24][:, _PERM24]
    w1pln = W1[:, base + 24:base + 32]
    w1nbl = W1[:, base + 32:base + 56][:, _PERM24]
    w1nbln = W1[:, base + 56:base + 64]
    w1nbgn = W1[:, base + 64:base + 72]
    w1gn = jnp.concatenate([w1pl, w1pln], axis=1)
    w1ge = jnp.concatenate([w1nbl, w1nbln, w1nbgn], axis=1)

    col = lambda v: v.reshape(-1, 1)
    return (hv2, x16, he2, idx_flat, wp_p, bp_c, w1n, w1e, w1nb, w1gn,
            w1ge, col(b1), W2, col(b2), W3, col(b3), Wd1, col(bd1),
            Wd2, col(bd2), col(g0), col(be0), col(g1), col(be1))


def kernel(h_V, h_E, E_idx, X, Wp, bp, W1, b1, W2, b2, W3, b3,
           Wd1, bd1, Wd2, bd2, g0, be0, g1, be1):
    (hv2, x16, he2, idx_flat, wp_p, bp_c, w1n, w1e, w1nb, w1gn, w1ge,
     b1c, w2, b2c, w3, b3c, wd1, bd1c, wd2, bd2c, g0c, be0c, g1c_,
     be1c) = _prep(h_V, h_E, E_idx, X, Wp, bp, W1, b1, W2, b2, W3, b3,
                   Wd1, bd1, Wd2, bd2, g0, be0, g1, be1)
    g1_rows, g2_rows = _sc_gather(hv2, x16, idx_flat)
    out = _tc_forward(hv2, x16, he2, g1_rows, g2_rows, wp_p, bp_c, w1n,
                      w1e, w1nb, w1gn, w1ge, b1c, w2, b2c, w3, b3c, wd1,
                      bd1c, wd2, bd2c, g0c, be0c, g1c_, be1c)
    return (out.reshape(B, L, ND), h_E)


# speedup vs baseline: 13.0885x; 1.0886x over previous
import functools

import jax
import jax.numpy as jnp
import numpy as np
from jax import lax
from jax.experimental import pallas as pl
from jax.experimental.pallas import tpu as pltpu
from jax.experimental.pallas import tpu_sc as plsc

B, L, K = 4, 1024, 32
ND, ED, HD, NP = 128, 128, 128, 8
POS_SCALE = 10.0
NIDX = B * L * K
BL = 128
E = BL * K
NBLK = (B * L) // BL

NC, NS = 2, 16
NW = NC * NS
NPW = NIDX // NW
CHUNK = 128
NCHUNK = NPW // CHUNK


def _sc_gather_body(hv_hbm, x_hbm, idx_hbm, g1_hbm, g2_hbm,
                    idx_v, rows1_v, rows2_v, sem1, sem2):
    cid = lax.axis_index("c")
    sid = lax.axis_index("s")
    wid = sid * NC + cid
    base = wid * NPW
    batch_off = (base // (L * K)) * L

    def chunk(i, carry):
        off = base + i * CHUNK
        pltpu.sync_copy(idx_hbm.at[pl.ds(off, CHUNK)], idx_v)
        for j in range(CHUNK // 16):
            sl = pl.ds(j * 16, 16)
            idx_v[sl] = idx_v[sl] + batch_off
        cp1 = pltpu.async_copy(hv_hbm.at[idx_v], rows1_v, sem1)
        cp2 = pltpu.async_copy(x_hbm.at[idx_v], rows2_v, sem2)
        cp1.wait()
        cp2.wait()
        pltpu.sync_copy(rows1_v, g1_hbm.at[pl.ds(off, CHUNK)])
        pltpu.sync_copy(rows2_v, g2_hbm.at[pl.ds(off, CHUNK)])
        return carry

    lax.fori_loop(0, NCHUNK, chunk, 0)


def _sc_gather(hv2, x16, idx_flat):
    mesh = plsc.VectorSubcoreMesh(core_axis_name="c", subcore_axis_name="s")
    fn = pl.kernel(
        _sc_gather_body,
        out_type=[
            jax.ShapeDtypeStruct((NIDX, ND), jnp.float32),
            jax.ShapeDtypeStruct((NIDX, 16), jnp.float32),
        ],
        mesh=mesh,
        compiler_params=pltpu.CompilerParams(use_tc_tiling_on_sc=False),
        scratch_types=[
            pltpu.VMEM((CHUNK,), jnp.int32),
            pltpu.VMEM((CHUNK, ND), jnp.float32),
            pltpu.VMEM((CHUNK, 16), jnp.float32),
            pltpu.SemaphoreType.DMA,
            pltpu.SemaphoreType.DMA,
        ],
    )
    return fn(hv2, x16, idx_flat)


def _nt(a, b):
    return lax.dot_general(a, b, (((1,), (1,)), ((), ())),
                           preferred_element_type=jnp.float32)


def _nn(a, b):
    return lax.dot_general(a, b, (((1,), (0,)), ((), ())),
                           preferred_element_type=jnp.float32)


def _tn(a, b):
    return lax.dot_general(a, b, (((0,), (0,)), ((), ())),
                           preferred_element_type=jnp.float32)


def _frames_t(xT):
    n_ = [xT[i:i + 1] for i in range(3)]
    ca = [xT[i:i + 1] for i in range(3, 6)]
    c_ = [xT[i:i + 1] for i in range(6, 9)]
    v1 = [c_[i] - ca[i] for i in range(3)]
    v2 = [n_[i] - ca[i] for i in range(3)]
    n1 = jnp.sqrt(v1[0] * v1[0] + v1[1] * v1[1] + v1[2] * v1[2])
    r1 = 1.0 / (n1 + 1e-8)
    e1 = [v1[i] * r1 for i in range(3)]
    d = e1[0] * v2[0] + e1[1] * v2[1] + e1[2] * v2[2]
    u2 = [v2[i] - e1[i] * d for i in range(3)]
    n2 = jnp.sqrt(u2[0] * u2[0] + u2[1] * u2[1] + u2[2] * u2[2])
    r2 = 1.0 / (n2 + 1e-8)
    e2 = [u2[i] * r2 for i in range(3)]
    e3 = [e1[1] * e2[2] - e1[2] * e2[1],
          e1[2] * e2[0] - e1[0] * e2[2],
          e1[0] * e2[1] - e1[1] * e2[0]]
    t = [ca[i] * (1.0 / POS_SCALE) for i in range(3)]
    return e1, e2, e3, t


def _layer_norm_t(x, g, b):
    mu = jnp.mean(x, axis=0, keepdims=True)
    var = jnp.mean((x - mu) ** 2, axis=0, keepdims=True)
    return (x - mu) / jnp.sqrt(var + 1e-5) * g + b


def _tc_body(hv_ref, x_ref, he_ref, g1_ref, g2_ref,
             wp_ref, bp_ref, w1n_ref, w1e_ref, w1nb_ref, w1gn_ref,
             w1ge_ref, b1_ref, w2_ref, b2_ref, w3_ref, b3_ref,
             wd1_ref, bd1_ref, wd2_ref, bd2_ref,
             g0_ref, be0_ref, g1w_ref, be1_ref,
             i16_ref, i128_ref, out_ref):
    bf = jnp.bfloat16
    hv = hv_ref[...]
    hv_b = hv.astype(bf)
    i16 = i16_ref[...]
    i128 = i128_ref[...]

    xT = _nt(i16, x_ref[...])
    e1, e2, e3, t = _frames_t(xT)
    plT = _nt(wp_ref[...], hv_b) + bp_ref[...]
    plx, ply, plz = plT[0:8], plT[8:16], plT[16:24]
    pln = jnp.sqrt(plx * plx + ply * ply + plz * plz + 1e-8)
    pg = [e1[i] * plx + e2[i] * ply + e3[i] * plz + t[i] for i in range(3)]
    nodegeo = jnp.concatenate([plx, ply, plz, pln], axis=0)
    aT = (_nt(w1n_ref[...], hv_b) + _nn(w1gn_ref[...], nodegeo.astype(bf))
          + b1_ref[...])

    dmat = jnp.concatenate(pg + t + e1 + e2 + e3, axis=0)

    il = lax.broadcasted_iota(jnp.int32, (BL, E), 0)
    ie = lax.broadcasted_iota(jnp.int32, (BL, E), 1)
    sel = jnp.where(lax.shift_right_logical(ie, 5) == il, 1.0, 0.0)
    sel = sel.astype(jnp.bfloat16)

    de = _nn(dmat.astype(bf), sel)
    pgxe, pgye, pgze = de[0:8], de[8:16], de[16:24]
    txe, tye, tze = de[24:25], de[25:26], de[26:27]
    e1xe, e1ye, e1ze = de[27:28], de[28:29], de[29:30]
    e2xe, e2ye, e2ze = de[30:31], de[31:32], de[32:33]
    e3xe, e3ye, e3ze = de[33:34], de[34:35], de[35:36]

    g1_b = g1_ref[...].astype(bf)
    g2T = _nt(i16, g2_ref[...])
    ne1, ne2, ne3, nt_ = _frames_t(g2T)
    nplT = _nt(wp_ref[...], g1_b) + bp_ref[...]
    npx, npy, npz = nplT[0:8], nplT[8:16], nplT[16:24]
    pgn = [ne1[i] * npx + ne2[i] * npy + ne3[i] * npz + nt_[i]
           for i in range(3)]

    dx = pgn[0] - txe
    dy = pgn[1] - tye
    dz = pgn[2] - tze
    nblx = e1xe * dx + e1ye * dy + e1ze * dz
    nbly = e2xe * dx + e2ye * dy + e2ze * dz
    nblz = e3xe * dx + e3ye * dy + e3ze * dz
    nbln = jnp.sqrt(nblx * nblx + nbly * nbly + nblz * nblz + 1e-8)
    gx = pgxe - pgn[0]
    gy = pgye - pgn[1]
    gz = pgze - pgn[2]
    nbgn = jnp.sqrt(gx * gx + gy * gy + gz * gz + 1e-8)
    geomT = jnp.concatenate([nblx, nbly, nblz, nbln, nbgn], axis=0)

    h1 = _nt(w1e_ref[...], he_ref[...].astype(bf))
    h1 = h1 + _nt(w1nb_ref[...], g1_b)
    h1 = h1 + _nn(w1ge_ref[...], geomT.astype(bf))
    h1 = h1 + _nn(aT.astype(bf), sel)
    h1 = jnp.maximum(h1, 0.0).astype(bf)
    h2 = jnp.maximum(_nn(w2_ref[...], h1) + b2_ref[...], 0.0).astype(bf)
    mT = _nn(w3_ref[...], h2) + b3_ref[...]

    meanT = _nt(mT.astype(bf), sel) * (1.0 / K)

    hvT = _nt(i128, hv)
    r0 = _layer_norm_t(hvT + meanT, g0_ref[...], be0_ref[...])
    d1 = jnp.maximum(_nn(wd1_ref[...], r0) + bd1_ref[...], 0.0)
    dm = _nn(wd2_ref[...], d1) + bd2_ref[...]
    r1 = _layer_norm_t(r0 + dm, g1w_ref[...], be1_ref[...])

    out_ref[...] = _tn(r1, i128)


def _tc_forward(hv2, x16, he2, g1, g2, wp_p, bp_c, w1n, w1e, w1nb, w1gn,
                w1ge, b1c, w2, b2c, w3, b3c, wd1, bd1c, wd2, bd2c,
                g0c, be0c, g1c_, be1c, interpret=False):
    i16 = jnp.eye(16, dtype=jnp.float32)
    i128 = jnp.eye(128, dtype=jnp.float32)

    def row_spec(r, c):
        return pl.BlockSpec((r, c), lambda i: (i, 0))

    def full_spec(shape):
        return pl.BlockSpec(shape, lambda i: tuple(0 for _ in shape))

    in_specs = [
        row_spec(BL, ND),
        row_spec(BL, 16),
        row_spec(E, ED),
        row_spec(E, ND),
        row_spec(E, 16),
        full_spec((24, ND)),
        full_spec((24, 1)),
        full_spec((HD, ND)),
        full_spec((HD, ED)),
        full_spec((HD, ND)),
        full_spec((HD, 32)),
        full_spec((HD, 40)),
        full_spec((HD, 1)),
        full_spec((HD, HD)),
        full_spec((HD, 1)),
        full_spec((HD, HD)),
        full_spec((HD, 1)),
        full_spec((4 * HD, HD)),
        full_spec((4 * HD, 1)),
        full_spec((HD, 4 * HD)),
        full_spec((HD, 1)),
        full_spec((HD, 1)),
        full_spec((HD, 1)),
        full_spec((HD, 1)),
        full_spec((HD, 1)),
        full_spec((16, 16)),
        full_spec((ND, ND)),
    ]
    out = pl.pallas_call(
        _tc_body,
        grid=(NBLK,),
        in_specs=in_specs,
        out_specs=pl.BlockSpec((BL, ND), lambda i: (i, 0)),
        out_shape=jax.ShapeDtypeStruct((B * L, ND), jnp.float32),
        interpret=interpret,
    )(hv2, x16, he2, g1, g2, wp_p, bp_c, w1n, w1e, w1nb, w1gn, w1ge,
      b1c, w2, b2c, w3, b3c, wd1, bd1c, wd2, bd2c, g0c, be0c, g1c_,
      be1c, i16, i128)
    return out


_PERM24 = np.array([3 * p + c for c in range(3) for p in range(NP)])


def _prep(h_V, h_E, E_idx, X, Wp, bp, W1, b1, W2, b2, W3, b3,
          Wd1, bd1, Wd2, bd2, g0, be0, g1, be1):
    hv2 = h_V.reshape(B * L, ND)
    x9 = X[:, :, :3, :].reshape(B * L, 9)
    x16 = jnp.concatenate(
        [x9, jnp.zeros((B * L, 7), jnp.float32)], axis=1)
    he2 = h_E.reshape(NIDX, ED)
    idx_flat = E_idx.reshape(NIDX).astype(jnp.int32)

    wp_p = Wp[_PERM24, :]
    bp_c = bp[_PERM24].reshape(24, 1)
    w1n = W1[:, 0:ND]
    w1e = W1[:, ND:ND + ED]
    w1nb = W1[:, ND + ED:2 * ND + ED]
    base = 2 * ND + ED
    w1pl = W1[:, base:base + 24][:, _PERM24]
    w1pln = W1[:, base + 24:base + 32]
    w1nbl = W1[:, base + 32:base + 56][:, _PERM24]
    w1nbln = W1[:, base + 56:base + 64]
    w1nbgn = W1[:, base + 64:base + 72]
    w1gn = jnp.concatenate([w1pl, w1pln], axis=1)
    w1ge = jnp.concatenate([w1nbl, w1nbln, w1nbgn], axis=1)

    bf = jnp.bfloat16
    wp_p, w1n, w1e, w1nb, w1gn, w1ge, W2, W3 = (
        a.astype(bf) for a in (wp_p, w1n, w1e, w1nb, w1gn, w1ge, W2, W3))

    col = lambda v: v.reshape(-1, 1)
    return (hv2, x16, he2, idx_flat, wp_p, bp_c, w1n, w1e, w1nb, w1gn,
            w1ge, col(b1), W2, col(b2), W3, col(b3), Wd1, col(bd1),
            Wd2, col(bd2), col(g0), col(be0), col(g1), col(be1))


def kernel(h_V, h_E, E_idx, X, Wp, bp, W1, b1, W2, b2, W3, b3,
           Wd1, bd1, Wd2, bd2, g0, be0, g1, be1):
    (hv2, x16, he2, idx_flat, wp_p, bp_c, w1n, w1e, w1nb, w1gn, w1ge,
     b1c, w2, b2c, w3, b3c, wd1, bd1c, wd2, bd2c, g0c, be0c, g1c_,
     be1c) = _prep(h_V, h_E, E_idx, X, Wp, bp, W1, b1, W2, b2, W3, b3,
                   Wd1, bd1, Wd2, bd2, g0, be0, g1, be1)
    g1_rows, g2_rows = _sc_gather(hv2, x16, idx_flat)
    out = _tc_forward(hv2, x16, he2, g1_rows, g2_rows, wp_p, bp_c, w1n,
                      w1e, w1nb, w1gn, w1ge, b1c, w2, b2c, w3, b3c, wd1,
                      bd1c, wd2, bd2c, g0c, be0c, g1c_, be1c)
    return (out.reshape(B, L, ND), h_E)
